# baseline (speedup 1.0000x reference)
import jax
import jax.numpy as jnp
from jax.experimental import pallas as pl

_BLOCK_M = 1024


def _router_block(x_ref, w_ref, b_ref, o_ref):
    o_ref[...] = (
        jnp.dot(x_ref[...], w_ref[...], preferred_element_type=jnp.float32)
        + b_ref[...]
    )


def kernel(hidden_states, weight, bias):
    b, s, h = hidden_states.shape
    e = weight.shape[0]
    m = b * s
    x = hidden_states.reshape(m, h)
    w_t = weight.T
    bias2 = bias.reshape(1, e)

    block_m = min(_BLOCK_M, m)
    grid = (m // block_m,)
    out = pl.pallas_call(
        _router_block,
        grid=grid,
        in_specs=[
            pl.BlockSpec((block_m, h), lambda i: (i, 0)),
            pl.BlockSpec((h, e), lambda i: (0, 0)),
            pl.BlockSpec((1, e), lambda i: (0, 0)),
        ],
        out_specs=pl.BlockSpec((block_m, e), lambda i: (i, 0)),
        out_shape=jax.ShapeDtypeStruct((m, e), jnp.float32),
    )(x, w_t, bias2)
    return out


# speedup vs baseline: 1.0176x; 1.0176x over previous
import jax
import jax.numpy as jnp
from jax import lax
from jax.experimental import pallas as pl
from jax.experimental.pallas import tpu as pltpu

_BLOCK_M = 1024


def _router_block(x_ref, w_ref, b_ref, o_ref):
    o_ref[...] = (
        lax.dot_general(
            x_ref[...],
            w_ref[...],
            (((1,), (1,)), ((), ())),
            preferred_element_type=jnp.float32,
        )
        + b_ref[...]
    )


def kernel(hidden_states, weight, bias):
    b, s, h = hidden_states.shape
    e = weight.shape[0]
    m = b * s
    x = hidden_states.reshape(m, h)
    bias2 = bias.reshape(1, e)

    block_m = min(_BLOCK_M, m)
    grid = (m // block_m,)
    out = pl.pallas_call(
        _router_block,
        grid=grid,
        in_specs=[
            pl.BlockSpec((block_m, h), lambda i: (i, 0)),
            pl.BlockSpec((e, h), lambda i: (0, 0)),
            pl.BlockSpec((1, e), lambda i: (0, 0)),
        ],
        out_specs=pl.BlockSpec((block_m, e), lambda i: (i, 0)),
        out_shape=jax.ShapeDtypeStruct((m, e), jnp.float32),
        compiler_params=pltpu.CompilerParams(
            dimension_semantics=("arbitrary",),
            vmem_limit_bytes=110 * 1024 * 1024,
        ),
    )(x, weight, bias2)
    return out


# speedup vs baseline: 1.0198x; 1.0021x over previous
import jax
import jax.numpy as jnp
from jax import lax
from jax.experimental import pallas as pl
from jax.experimental.pallas import tpu as pltpu

_SPLIT = 4
_BLOCK_M = 256


def _router_block(*refs):
    x_refs = refs[:_SPLIT]
    w_ref, b_ref = refs[_SPLIT], refs[_SPLIT + 1]
    o_ref = refs[_SPLIT + 2]
    w = w_ref[...]
    b = b_ref[...]
    for k, x_ref in enumerate(x_refs):
        o_ref[k * _BLOCK_M:(k + 1) * _BLOCK_M, :] = (
            lax.dot_general(
                x_ref[...],
                w,
                (((1,), (1,)), ((), ())),
                preferred_element_type=jnp.float32,
            )
            + b
        )


def kernel(hidden_states, weight, bias):
    b, s, h = hidden_states.shape
    e = weight.shape[0]
    m = b * s
    x = hidden_states.reshape(m, h)
    bias2 = bias.reshape(1, e)

    span = _BLOCK_M * _SPLIT
    steps = m // span

    def x_map(k):
        return lambda i: (i * _SPLIT + k, 0)

    in_specs = [
        pl.BlockSpec((_BLOCK_M, h), x_map(k)) for k in range(_SPLIT)
    ] + [
        pl.BlockSpec((e, h), lambda i: (0, 0)),
        pl.BlockSpec((1, e), lambda i: (0, 0)),
    ]
    out = pl.pallas_call(
        _router_block,
        grid=(steps,),
        in_specs=in_specs,
        out_specs=pl.BlockSpec((span, e), lambda i: (i, 0)),
        out_shape=jax.ShapeDtypeStruct((m, e), jnp.float32),
        compiler_params=pltpu.CompilerParams(
            dimension_semantics=("arbitrary",),
        ),
    )(*([x] * _SPLIT), weight, bias2)
    return out
